# baseline (speedup 1.0000x reference)
import functools

import jax
import jax.numpy as jnp
from jax import lax
from jax.experimental import pallas as pl
from jax.experimental.pallas import tpu as pltpu
from jax.experimental.pallas import tpu_sc as plsc

_B = 1024
_D = 768
_T = 2
_G = 4
_E = 8
_NCLS = 4
_BLK = 256
_NBLK = 16
_P = _NBLK * _BLK
_NW = 32
_CLS_PAD = 128

_bf = jnp.bfloat16
_f32 = jnp.float32


def _b2(x):
    return x.astype(_bf)


def _dot(a, b):
    return jnp.dot(_b2(a), _b2(b), preferred_element_type=_f32)


def _make_tri():
    r = jax.lax.broadcasted_iota(jnp.int32, (_B, _B), 0)
    c = jax.lax.broadcasted_iota(jnp.int32, (_B, _B), 1)
    return (c < r).astype(_bf)


def _route_core(logits, xin, tri, xin_ref, pos0_ref, pos1_ref, w0_ref,
                w1_ref, be_ref):
    xin_ref[...] = xin
    iota_e = jax.lax.broadcasted_iota(jnp.int32, (_B, _E), 1)
    m0 = jnp.max(logits, axis=1, keepdims=True)
    i0 = jnp.min(jnp.where(logits == m0, iota_e, _E), axis=1, keepdims=True)
    sel0 = iota_e == i0
    masked = jnp.where(sel0, -jnp.inf, logits)
    m1 = jnp.max(masked, axis=1, keepdims=True)
    i1 = jnp.min(jnp.where(masked == m1, iota_e, _E), axis=1, keepdims=True)
    sel1 = iota_e == i1
    e1 = jnp.exp(m1 - m0)
    s = 1.0 + e1
    w0_ref[...] = 1.0 / s
    w1_ref[...] = e1 / s

    onehot = (sel0 | sel1).astype(_f32)
    ranks = jnp.dot(tri, _b2(onehot), preferred_element_type=_f32)
    counts = jnp.sum(onehot, axis=0, keepdims=True)
    pc = jnp.ceil(counts * (1.0 / _BLK)) * _BLK
    ue = jax.lax.broadcasted_iota(jnp.int32, (_E, _E), 0)
    uc = jax.lax.broadcasted_iota(jnp.int32, (_E, _E), 1)
    upper = (ue < uc).astype(_bf)
    base = jnp.dot(_b2(pc), upper, preferred_element_type=_f32)
    slot = base + ranks
    pos0_ref[...] = jnp.sum(jnp.where(sel0, slot, 0.0), axis=1,
                            keepdims=True).astype(jnp.int32)
    pos1_ref[...] = jnp.sum(jnp.where(sel1, slot, 0.0), axis=1,
                            keepdims=True).astype(jnp.int32)
    bs = base * (1.0 / _BLK)
    bvec = jax.lax.broadcasted_iota(jnp.int32, (_NBLK, 1), 0).astype(_f32)
    be_ref[...] = (jnp.sum((bs <= bvec).astype(_f32), axis=1, keepdims=True)
                   - 1.0).astype(jnp.int32)


def _gates4_body(emb_ref, gw0, gb0, gw1, gb1, gw2, gb2, gw3, gb3,
                 *out_refs):
    gws = (gw0, gw1, gw2, gw3)
    gbs = (gb0, gb1, gb2, gb3)
    tri = _make_tri()
    for g in range(_G):
        x0 = emb_ref[:, 2 * g, :]
        x1 = emb_ref[:, 2 * g + 1, :]
        xin = (x0 + x1) * 0.5
        gw = gws[g][...]
        logits = (_dot(x0, gw[0:_D]) + _dot(x1, gw[_D:2 * _D])
                  + gbs[g][...])
        _route_core(logits, xin, tri, *out_refs[g * 6:(g + 1) * 6])


def _gates_clf_body(g00, g10, g01, g11, g02, g12, g03, g13,
                    w00, w10, w01, w11, w02, w12, w03, w13,
                    gw_ref, gb_ref, *out_refs):
    ogs = [w00[...] * g00[...] + w10[...] * g10[...],
           w01[...] * g01[...] + w11[...] * g11[...],
           w02[...] * g02[...] + w12[...] * g12[...],
           w03[...] * g03[...] + w13[...] * g13[...]]
    xin = (((ogs[0] + ogs[1]) + ogs[2]) + ogs[3]) * 0.25
    gw = gw_ref[...]
    logits = _dot(ogs[0], gw[0:_D]) + gb_ref[...]
    for g in range(1, _G):
        logits = logits + _dot(ogs[g], gw[g * _D:(g + 1) * _D])
    _route_core(logits, xin, _make_tri(), *out_refs)


_GATE_OUT = (
    jax.ShapeDtypeStruct((_B, _D), _f32),
    jax.ShapeDtypeStruct((_B, 1), jnp.int32),
    jax.ShapeDtypeStruct((_B, 1), jnp.int32),
    jax.ShapeDtypeStruct((_B, 1), _f32),
    jax.ShapeDtypeStruct((_B, 1), _f32),
    jax.ShapeDtypeStruct((_NBLK, 1), jnp.int32),
)


def _gates4(embeddings, groups):
    args = [embeddings]
    for p in groups:
        args.append(p["gate_W"])
        args.append(p["gate_b"].reshape(1, _E))
    outs = pl.pallas_call(_gates4_body, out_shape=_GATE_OUT * _G)(*args)
    return [outs[g * 6:(g + 1) * 6] for g in range(_G)]


def _gates_clf(gpairs, wpairs, gw, gb):
    return pl.pallas_call(_gates_clf_body, out_shape=_GATE_OUT)(
        *gpairs, *wpairs, gw, gb.reshape(1, _E))


def _final_body(f0_ref, f1_ref, w0_ref, w1_ref, o_ref):
    f = w0_ref[...] * f0_ref[...] + w1_ref[...] * f1_ref[...]
    o_ref[...] = f[:, :_NCLS]


def _final(f0, f1, w0, w1):
    return pl.pallas_call(
        _final_body,
        out_shape=jax.ShapeDtypeStruct((_B, _NCLS), _f32),
    )(f0, f1, w0, w1)


_SC_CACHE = {}


def _sc_mesh():
    return plsc.VectorSubcoreMesh(core_axis_name="c", subcore_axis_name="s")


_TOK_W = _B // _NW


def _sc_dispatch(*args):
    if "dispatch" not in _SC_CACHE:
        _SC_CACHE["dispatch"] = functools.partial(
            pl.kernel,
            mesh=_sc_mesh(),
            out_type=jax.ShapeDtypeStruct((_P, _D), _f32),
            scratch_types=[
                pltpu.VMEM((_TOK_W,), jnp.int32),
                pltpu.VMEM((_TOK_W, _D), _f32),
            ],
        )(_sc_dispatch_body)
    return _SC_CACHE["dispatch"](*args)


def _sc_dispatch_body(pos0_hbm, pos1_hbm, xin_hbm, xdisp_hbm,
                      p_v, rows_v):
    cid = lax.axis_index("c")
    sid = lax.axis_index("s")
    wid = sid * 2 + cid
    t0 = wid * _TOK_W
    pltpu.sync_copy(xin_hbm.at[pl.ds(t0, _TOK_W)], rows_v)
    pltpu.sync_copy(pos0_hbm.at[pl.ds(t0, _TOK_W)], p_v)
    pltpu.sync_copy(rows_v, xdisp_hbm.at[p_v])
    pltpu.sync_copy(pos1_hbm.at[pl.ds(t0, _TOK_W)], p_v)
    pltpu.sync_copy(rows_v, xdisp_hbm.at[p_v])


def _sc_dispatch4(*args):
    if "dispatch4" not in _SC_CACHE:
        _SC_CACHE["dispatch4"] = functools.partial(
            pl.kernel,
            mesh=_sc_mesh(),
            out_type=tuple(jax.ShapeDtypeStruct((_P, _D), _f32)
                           for _ in range(_G)),
            scratch_types=[
                pltpu.VMEM((_TOK_W,), jnp.int32),
                pltpu.VMEM((_TOK_W, _D), _f32),
            ],
        )(_sc_dispatch4_body)
    return _SC_CACHE["dispatch4"](*args)


def _sc_dispatch4_body(p00, p10, p01, p11, p02, p12, p03, p13,
                       x0, x1, x2, x3, d0, d1, d2, d3, p_v, rows_v):
    cid = lax.axis_index("c")
    sid = lax.axis_index("s")
    wid = sid * 2 + cid
    t0 = wid * _TOK_W
    ps = ((p00, p10), (p01, p11), (p02, p12), (p03, p13))
    xs = (x0, x1, x2, x3)
    ds = (d0, d1, d2, d3)
    for g in range(_G):
        pltpu.sync_copy(xs[g].at[pl.ds(t0, _TOK_W)], rows_v)
        for k in range(2):
            pltpu.sync_copy(ps[g][k].at[pl.ds(t0, _TOK_W)], p_v)
            pltpu.sync_copy(rows_v, ds[g].at[p_v])


_CTOK = _B // 8


def _sc_combine4(*args):
    if "combine4" not in _SC_CACHE:
        _SC_CACHE["combine4"] = functools.partial(
            pl.kernel,
            mesh=_sc_mesh(),
            out_type=tuple(jax.ShapeDtypeStruct((_B, _D), _f32)
                           for _ in range(8)),
            scratch_types=[
                pltpu.VMEM((_TOK_W,), jnp.int32),
                pltpu.VMEM((_TOK_W,), jnp.int32),
                pltpu.VMEM((_TOK_W, _D), _f32),
                pltpu.VMEM((_TOK_W, _D), _f32),
                pltpu.SemaphoreType.DMA,
                pltpu.SemaphoreType.DMA,
            ],
        )(_sc_combine4_body)
    return _SC_CACHE["combine4"](*args)


def _sc_combine4_body(y0, y1, y2, y3,
                      p00, p10, p01, p11, p02, p12, p03, p13,
                      g00, g10, g01, g11, g02, g12, g03, g13,
                      pp_a, pp_b, rows_a, rows_b, sem_a, sem_b):
    cid = lax.axis_index("c")
    sid = lax.axis_index("s")
    wid = sid * 2 + cid
    t0 = wid * _TOK_W
    ys = (y0, y1, y2, y3)
    ps = ((p00, p10), (p01, p11), (p02, p12), (p03, p13))
    gs = ((g00, g10), (g01, g11), (g02, g12), (g03, g13))
    jobs = [(ps[g][k], ys[g], gs[g][k]) for g in range(_G) for k in range(2)]
    pps = (pp_a, pp_b)
    bufs = (rows_a, rows_b)
    sems = (sem_a, sem_b)
    pending = None
    for j, (p, y, gdst) in enumerate(jobs):
        b = j % 2
        pltpu.sync_copy(p.at[pl.ds(t0, _TOK_W)], pps[b])
        c = pltpu.async_copy(y.at[pps[b]], bufs[b], sems[b])
        if pending is not None:
            pc, pb, pdst = pending
            pc.wait()
            pltpu.sync_copy(bufs[pb], pdst.at[pl.ds(t0, _TOK_W)])
        pending = (c, b, gdst)
    pc, pb, pdst = pending
    pc.wait()
    pltpu.sync_copy(bufs[pb], pdst.at[pl.ds(t0, _TOK_W)])


def _sc_combine_clf(*args):
    if "combine_clf" not in _SC_CACHE:
        _SC_CACHE["combine_clf"] = functools.partial(
            pl.kernel,
            mesh=_sc_mesh(),
            out_type=(jax.ShapeDtypeStruct((_B, _CLS_PAD), _f32),
                      jax.ShapeDtypeStruct((_B, _CLS_PAD), _f32)),
            scratch_types=[
                pltpu.VMEM((_TOK_W,), jnp.int32),
                pltpu.VMEM((_TOK_W, _CLS_PAD), _f32),
                pltpu.SemaphoreType.DMA,
            ],
        )(_sc_combine_clf_body)
    return _SC_CACHE["combine_clf"](*args)


def _sc_combine_clf_body(y_hbm, p0_hbm, p1_hbm, f0_hbm, f1_hbm,
                         pp_v, rows_v, sem):
    cid = lax.axis_index("c")
    sid = lax.axis_index("s")
    wid = sid * 2 + cid
    t0 = wid * _TOK_W
    for p_hbm, f_hbm in ((p0_hbm, f0_hbm), (p1_hbm, f1_hbm)):
        pltpu.sync_copy(p_hbm.at[pl.ds(t0, _TOK_W)], pp_v)
        pltpu.async_copy(y_hbm.at[pp_v], rows_v, sem).wait()
        pltpu.sync_copy(rows_v, f_hbm.at[pl.ds(t0, _TOK_W)])


def _gemm_body(be_ref, x_ref, w1_ref, b1_ref, w2_ref, b2_ref,
               w3_ref, b3_ref, w4_ref, b4_ref, y_ref):
    del be_ref
    x = x_ref[...]
    h = jax.nn.relu(_dot(x, w1_ref[0]) + b1_ref[0])
    h = jax.nn.relu(_dot(h, w2_ref[0]) + b2_ref[0])
    h = jax.nn.relu(_dot(h, w3_ref[0]) + b3_ref[0])
    y_ref[...] = _dot(h, w4_ref[0]) + b4_ref[0]


def _grouped_gemm(be, xdisp_bf, mlp, out_dim):
    (w1, b1), (w2, b2), (w3, b3), (w4, b4) = mlp
    h1, h2, h3 = w1.shape[2], w2.shape[2], w3.shape[2]
    exp = lambda shape: pl.BlockSpec(
        (1,) + shape, lambda i, be: (be[i],) + (0,) * len(shape))
    grid_spec = pltpu.PrefetchScalarGridSpec(
        num_scalar_prefetch=1,
        grid=(_NBLK,),
        in_specs=[
            pl.BlockSpec((_BLK, _D), lambda i, be: (i, 0)),
            exp((_D, h1)), exp((1, h1)),
            exp((h1, h2)), exp((1, h2)),
            exp((h2, h3)), exp((1, h3)),
            exp((h3, out_dim)), exp((1, out_dim)),
        ],
        out_specs=pl.BlockSpec((_BLK, out_dim), lambda i, be: (i, 0)),
    )
    return pl.pallas_call(
        _gemm_body,
        grid_spec=grid_spec,
        out_shape=jax.ShapeDtypeStruct((_P, out_dim), _f32),
    )(be, xdisp_bf,
      w1, b1.reshape(_E, 1, h1), w2, b2.reshape(_E, 1, h2),
      w3, b3.reshape(_E, 1, h3), w4, b4.reshape(_E, 1, out_dim))


def _dispatch_and_gemm(gouts, mlp, out_dim):
    xin, pos0, pos1, w0, w1, be = gouts
    pos0f = pos0.reshape(_B)
    pos1f = pos1.reshape(_B)
    xdisp = _sc_dispatch(pos0f, pos1f, xin)
    y = _grouped_gemm(be.reshape(_NBLK), xdisp, mlp, out_dim)
    return y, pos0f, pos1f


def kernel(embeddings, params):
    all_gouts = _gates4(embeddings, params["groups"])
    p0s = [g[1].reshape(_B) for g in all_gouts]
    p1s = [g[2].reshape(_B) for g in all_gouts]
    xdisps = _sc_dispatch4(
        *[x for g in range(_G) for x in (p0s[g], p1s[g])],
        *[g[0] for g in all_gouts])
    ys = [_grouped_gemm(all_gouts[g][5].reshape(_NBLK), xdisps[g],
                        params["groups"][g]["mlp"], _D)
          for g in range(_G)]
    gpairs = _sc_combine4(*ys, *[x for g in range(_G)
                                 for x in (p0s[g], p1s[g])])
    wpairs = [x for g in range(_G)
              for x in (all_gouts[g][3], all_gouts[g][4])]

    clf = params["clf"]
    gouts = _gates_clf(gpairs, wpairs, clf["gate_W"], clf["gate_b"])
    (w1, b1), (w2, b2), (w3, b3), (w4, b4) = clf["mlp"]
    w4p = jnp.pad(w4, ((0, 0), (0, 0), (0, _CLS_PAD - _NCLS)))
    b4p = jnp.pad(b4, ((0, 0), (0, _CLS_PAD - _NCLS)))
    mlp_p = ((w1, b1), (w2, b2), (w3, b3), (w4p, b4p))
    yc, p0c, p1c = _dispatch_and_gemm(gouts, mlp_p, _CLS_PAD)
    f0, f1 = _sc_combine_clf(yc, p0c, p1c)
    return _final(f0, f1, gouts[3], gouts[4])


# speedup vs baseline: 1.0172x; 1.0172x over previous
import functools

import jax
import jax.numpy as jnp
from jax import lax
from jax.experimental import pallas as pl
from jax.experimental.pallas import tpu as pltpu
from jax.experimental.pallas import tpu_sc as plsc

_B = 1024
_D = 768
_T = 2
_G = 4
_E = 8
_NCLS = 4
_BLK = 256
_NBLK = 16
_P = _NBLK * _BLK
_NW = 32
_CLS_PAD = 128

_bf = jnp.bfloat16
_f32 = jnp.float32


def _b2(x):
    return x.astype(_bf)


def _dot(a, b):
    return jnp.dot(_b2(a), _b2(b), preferred_element_type=_f32)


def _make_tri():
    r = jax.lax.broadcasted_iota(jnp.int32, (_B, _B), 0)
    c = jax.lax.broadcasted_iota(jnp.int32, (_B, _B), 1)
    return (c < r).astype(_bf)


def _route_core(logits, xin, tri, xin_ref, pos0_ref, pos1_ref, w0_ref,
                w1_ref, be_ref):
    xin_ref[...] = xin
    iota_e = jax.lax.broadcasted_iota(jnp.int32, (_B, _E), 1)
    m0 = jnp.max(logits, axis=1, keepdims=True)
    i0 = jnp.min(jnp.where(logits == m0, iota_e, _E), axis=1, keepdims=True)
    sel0 = iota_e == i0
    masked = jnp.where(sel0, -jnp.inf, logits)
    m1 = jnp.max(masked, axis=1, keepdims=True)
    i1 = jnp.min(jnp.where(masked == m1, iota_e, _E), axis=1, keepdims=True)
    sel1 = iota_e == i1
    e1 = jnp.exp(m1 - m0)
    s = 1.0 + e1
    w0_ref[...] = 1.0 / s
    w1_ref[...] = e1 / s

    onehot = (sel0 | sel1).astype(_f32)
    ranks = jnp.dot(tri, _b2(onehot), preferred_element_type=_f32)
    counts = jnp.sum(onehot, axis=0, keepdims=True)
    pc = jnp.ceil(counts * (1.0 / _BLK)) * _BLK
    ue = jax.lax.broadcasted_iota(jnp.int32, (_E, _E), 0)
    uc = jax.lax.broadcasted_iota(jnp.int32, (_E, _E), 1)
    upper = (ue < uc).astype(_bf)
    base = jnp.dot(_b2(pc), upper, preferred_element_type=_f32)
    slot = base + ranks
    pos0_ref[...] = jnp.sum(jnp.where(sel0, slot, 0.0), axis=1,
                            keepdims=True).astype(jnp.int32)
    pos1_ref[...] = jnp.sum(jnp.where(sel1, slot, 0.0), axis=1,
                            keepdims=True).astype(jnp.int32)
    bs = base * (1.0 / _BLK)
    bvec = jax.lax.broadcasted_iota(jnp.int32, (_NBLK, 1), 0).astype(_f32)
    be_ref[...] = (jnp.sum((bs <= bvec).astype(_f32), axis=1, keepdims=True)
                   - 1.0).astype(jnp.int32)


def _gates4_body(emb_ref, gw0, gb0, gw1, gb1, gw2, gb2, gw3, gb3,
                 *out_refs):
    gws = (gw0, gw1, gw2, gw3)
    gbs = (gb0, gb1, gb2, gb3)
    tri = _make_tri()
    for g in range(_G):
        x0 = emb_ref[:, 2 * g, :]
        x1 = emb_ref[:, 2 * g + 1, :]
        xin = (x0 + x1) * 0.5
        gw = gws[g][...]
        logits = (_dot(x0, gw[0:_D]) + _dot(x1, gw[_D:2 * _D])
                  + gbs[g][...])
        _route_core(logits, xin, tri, *out_refs[g * 6:(g + 1) * 6])


def _gates_clf_body(g00, g10, g01, g11, g02, g12, g03, g13,
                    w00, w10, w01, w11, w02, w12, w03, w13,
                    gw_ref, gb_ref, *out_refs):
    ogs = [w00[...] * g00[...] + w10[...] * g10[...],
           w01[...] * g01[...] + w11[...] * g11[...],
           w02[...] * g02[...] + w12[...] * g12[...],
           w03[...] * g03[...] + w13[...] * g13[...]]
    xin = (((ogs[0] + ogs[1]) + ogs[2]) + ogs[3]) * 0.25
    gw = gw_ref[...]
    logits = _dot(ogs[0], gw[0:_D]) + gb_ref[...]
    for g in range(1, _G):
        logits = logits + _dot(ogs[g], gw[g * _D:(g + 1) * _D])
    _route_core(logits, xin, _make_tri(), *out_refs)


_GATE_OUT = (
    jax.ShapeDtypeStruct((_B, _D), _f32),
    jax.ShapeDtypeStruct((_B, 1), jnp.int32),
    jax.ShapeDtypeStruct((_B, 1), jnp.int32),
    jax.ShapeDtypeStruct((_B, 1), _f32),
    jax.ShapeDtypeStruct((_B, 1), _f32),
    jax.ShapeDtypeStruct((_NBLK, 1), jnp.int32),
)


def _gates4(embeddings, groups):
    args = [embeddings]
    for p in groups:
        args.append(p["gate_W"])
        args.append(p["gate_b"].reshape(1, _E))
    outs = pl.pallas_call(_gates4_body, out_shape=_GATE_OUT * _G)(*args)
    return [outs[g * 6:(g + 1) * 6] for g in range(_G)]


def _gates_clf(gpairs, wpairs, gw, gb):
    return pl.pallas_call(_gates_clf_body, out_shape=_GATE_OUT)(
        *gpairs, *wpairs, gw, gb.reshape(1, _E))


def _final_body(f0_ref, f1_ref, w0_ref, w1_ref, o_ref):
    f = w0_ref[...] * f0_ref[...] + w1_ref[...] * f1_ref[...]
    o_ref[...] = f[:, :_NCLS]


def _final(f0, f1, w0, w1):
    return pl.pallas_call(
        _final_body,
        out_shape=jax.ShapeDtypeStruct((_B, _NCLS), _f32),
    )(f0, f1, w0, w1)


_SC_CACHE = {}


def _sc_mesh():
    return plsc.VectorSubcoreMesh(core_axis_name="c", subcore_axis_name="s")


_TOK_W = _B // _NW


def _sc_dispatch(*args):
    if "dispatch" not in _SC_CACHE:
        _SC_CACHE["dispatch"] = functools.partial(
            pl.kernel,
            mesh=_sc_mesh(),
            out_type=jax.ShapeDtypeStruct((_P, _D), _f32),
            scratch_types=[
                pltpu.VMEM((_TOK_W,), jnp.int32),
                pltpu.VMEM((_TOK_W, _D), _f32),
            ],
        )(_sc_dispatch_body)
    return _SC_CACHE["dispatch"](*args)


def _sc_dispatch_body(pos0_hbm, pos1_hbm, xin_hbm, xdisp_hbm,
                      p_v, rows_v):
    cid = lax.axis_index("c")
    sid = lax.axis_index("s")
    wid = sid * 2 + cid
    t0 = wid * _TOK_W
    pltpu.sync_copy(xin_hbm.at[pl.ds(t0, _TOK_W)], rows_v)
    pltpu.sync_copy(pos0_hbm.at[pl.ds(t0, _TOK_W)], p_v)
    pltpu.sync_copy(rows_v, xdisp_hbm.at[p_v])
    pltpu.sync_copy(pos1_hbm.at[pl.ds(t0, _TOK_W)], p_v)
    pltpu.sync_copy(rows_v, xdisp_hbm.at[p_v])


_CTOK = _B // 8


def _sc_combine4(*args):
    if "combine4" not in _SC_CACHE:
        _SC_CACHE["combine4"] = functools.partial(
            pl.kernel,
            mesh=_sc_mesh(),
            out_type=tuple(jax.ShapeDtypeStruct((_B, _D), _f32)
                           for _ in range(8)),
            scratch_types=[
                pltpu.VMEM((_TOK_W,), jnp.int32),
                pltpu.VMEM((_TOK_W,), jnp.int32),
                pltpu.VMEM((_TOK_W, _D), _f32),
                pltpu.VMEM((_TOK_W, _D), _f32),
                pltpu.SemaphoreType.DMA,
                pltpu.SemaphoreType.DMA,
            ],
        )(_sc_combine4_body)
    return _SC_CACHE["combine4"](*args)


def _sc_combine4_body(y0, y1, y2, y3,
                      p00, p10, p01, p11, p02, p12, p03, p13,
                      g00, g10, g01, g11, g02, g12, g03, g13,
                      pp_a, pp_b, rows_a, rows_b, sem_a, sem_b):
    cid = lax.axis_index("c")
    sid = lax.axis_index("s")
    wid = sid * 2 + cid
    t0 = wid * _TOK_W
    ys = (y0, y1, y2, y3)
    ps = ((p00, p10), (p01, p11), (p02, p12), (p03, p13))
    gs = ((g00, g10), (g01, g11), (g02, g12), (g03, g13))
    jobs = [(ps[g][k], ys[g], gs[g][k]) for g in range(_G) for k in range(2)]
    pps = (pp_a, pp_b)
    bufs = (rows_a, rows_b)
    sems = (sem_a, sem_b)
    pending = None
    for j, (p, y, gdst) in enumerate(jobs):
        b = j % 2
        pltpu.sync_copy(p.at[pl.ds(t0, _TOK_W)], pps[b])
        c = pltpu.async_copy(y.at[pps[b]], bufs[b], sems[b])
        if pending is not None:
            pc, pb, pdst = pending
            pc.wait()
            pltpu.sync_copy(bufs[pb], pdst.at[pl.ds(t0, _TOK_W)])
        pending = (c, b, gdst)
    pc, pb, pdst = pending
    pc.wait()
    pltpu.sync_copy(bufs[pb], pdst.at[pl.ds(t0, _TOK_W)])


def _sc_combine_clf(*args):
    if "combine_clf" not in _SC_CACHE:
        _SC_CACHE["combine_clf"] = functools.partial(
            pl.kernel,
            mesh=_sc_mesh(),
            out_type=(jax.ShapeDtypeStruct((_B, _CLS_PAD), _f32),
                      jax.ShapeDtypeStruct((_B, _CLS_PAD), _f32)),
            scratch_types=[
                pltpu.VMEM((_TOK_W,), jnp.int32),
                pltpu.VMEM((_TOK_W, _CLS_PAD), _f32),
                pltpu.SemaphoreType.DMA,
            ],
        )(_sc_combine_clf_body)
    return _SC_CACHE["combine_clf"](*args)


def _sc_combine_clf_body(y_hbm, p0_hbm, p1_hbm, f0_hbm, f1_hbm,
                         pp_v, rows_v, sem):
    cid = lax.axis_index("c")
    sid = lax.axis_index("s")
    wid = sid * 2 + cid
    t0 = wid * _TOK_W
    for p_hbm, f_hbm in ((p0_hbm, f0_hbm), (p1_hbm, f1_hbm)):
        pltpu.sync_copy(p_hbm.at[pl.ds(t0, _TOK_W)], pp_v)
        pltpu.async_copy(y_hbm.at[pp_v], rows_v, sem).wait()
        pltpu.sync_copy(rows_v, f_hbm.at[pl.ds(t0, _TOK_W)])


def _gemm_body(be_ref, x_ref, w1_ref, b1_ref, w2_ref, b2_ref,
               w3_ref, b3_ref, w4_ref, b4_ref, y_ref):
    del be_ref
    x = x_ref[...]
    h = jax.nn.relu(_dot(x, w1_ref[0]) + b1_ref[0])
    h = jax.nn.relu(_dot(h, w2_ref[0]) + b2_ref[0])
    h = jax.nn.relu(_dot(h, w3_ref[0]) + b3_ref[0])
    y_ref[...] = _dot(h, w4_ref[0]) + b4_ref[0]


def _grouped_gemm(be, xdisp_bf, mlp, out_dim):
    (w1, b1), (w2, b2), (w3, b3), (w4, b4) = mlp
    h1, h2, h3 = w1.shape[2], w2.shape[2], w3.shape[2]
    exp = lambda shape: pl.BlockSpec(
        (1,) + shape, lambda i, be: (be[i],) + (0,) * len(shape))
    grid_spec = pltpu.PrefetchScalarGridSpec(
        num_scalar_prefetch=1,
        grid=(_NBLK,),
        in_specs=[
            pl.BlockSpec((_BLK, _D), lambda i, be: (i, 0)),
            exp((_D, h1)), exp((1, h1)),
            exp((h1, h2)), exp((1, h2)),
            exp((h2, h3)), exp((1, h3)),
            exp((h3, out_dim)), exp((1, out_dim)),
        ],
        out_specs=pl.BlockSpec((_BLK, out_dim), lambda i, be: (i, 0)),
    )
    return pl.pallas_call(
        _gemm_body,
        grid_spec=grid_spec,
        out_shape=jax.ShapeDtypeStruct((_P, out_dim), _f32),
    )(be, xdisp_bf,
      w1, b1.reshape(_E, 1, h1), w2, b2.reshape(_E, 1, h2),
      w3, b3.reshape(_E, 1, h3), w4, b4.reshape(_E, 1, out_dim))


def _dispatch_and_gemm(gouts, mlp, out_dim):
    xin, pos0, pos1, w0, w1, be = gouts
    pos0f = pos0.reshape(_B)
    pos1f = pos1.reshape(_B)
    xdisp = _sc_dispatch(pos0f, pos1f, xin)
    y = _grouped_gemm(be.reshape(_NBLK), xdisp, mlp, out_dim)
    return y, pos0f, pos1f


def kernel(embeddings, params):
    ys, p0s, p1s, w0s, w1s = [], [], [], [], []
    all_gouts = _gates4(embeddings, params["groups"])
    for g in range(_G):
        p = params["groups"][g]
        gouts = all_gouts[g]
        y, p0f, p1f = _dispatch_and_gemm(gouts, p["mlp"], _D)
        ys.append(y)
        p0s.append(p0f)
        p1s.append(p1f)
        w0s.append(gouts[3])
        w1s.append(gouts[4])
    gpairs = _sc_combine4(*ys, *[x for g in range(_G)
                                 for x in (p0s[g], p1s[g])])
    wpairs = [x for g in range(_G) for x in (w0s[g], w1s[g])]

    clf = params["clf"]
    gouts = _gates_clf(gpairs, wpairs, clf["gate_W"], clf["gate_b"])
    (w1, b1), (w2, b2), (w3, b3), (w4, b4) = clf["mlp"]
    w4p = jnp.pad(w4, ((0, 0), (0, 0), (0, _CLS_PAD - _NCLS)))
    b4p = jnp.pad(b4, ((0, 0), (0, _CLS_PAD - _NCLS)))
    mlp_p = ((w1, b1), (w2, b2), (w3, b3), (w4p, b4p))
    yc, p0c, p1c = _dispatch_and_gemm(gouts, mlp_p, _CLS_PAD)
    f0, f1 = _sc_combine_clf(yc, p0c, p1c)
    return _final(f0, f1, gouts[3], gouts[4])
